# baseline (speedup 1.0000x reference)
import functools

import jax
import jax.numpy as jnp
from jax import lax
from jax.experimental import pallas as pl
from jax.experimental.pallas import tpu as pltpu
from jax.experimental.pallas import tpu_sc as plsc

NUM_ROWS = 1_000_000
DIM = 64
B, T = 4096, 200

NC, NS = 2, 16
NW = NC * NS
RT_FULL = NUM_ROWS // 128
A_ITERS = 246
PACKED_ROWS = NUM_ROWS // 2
TAIL0 = RT_FULL * 128

_mesh = plsc.VectorSubcoreMesh(core_axis_name="c", subcore_axis_name="s")
_params = pltpu.CompilerParams(use_tc_tiling_on_sc=True,
                               needs_layout_passes=False)


def _iota16():
    return jax.lax.iota(jnp.int32, 16)


@functools.partial(
    pl.kernel,
    mesh=_mesh,
    compiler_params=_params,
    out_type=jax.ShapeDtypeStruct((PACKED_ROWS, 128), jnp.float32),
    scratch_types=[
        pltpu.VMEM((64, 128), jnp.float32),
        pltpu.VMEM((64, 128), jnp.float32),
        pltpu.VMEM((64, 128), jnp.float32),
        pltpu.VMEM((64, 128), jnp.float32),
        pltpu.SemaphoreType.DMA,
        pltpu.SemaphoreType.DMA,
        pltpu.SemaphoreType.DMA,
        pltpu.SemaphoreType.DMA,
    ],
)
def _pack_kernel(tv_hbm, tail_hbm, out_hbm, s0, s1, o0, o1,
                 sem_i0, sem_i1, sem_o0, sem_o1):
    wid = lax.axis_index("s") * NC + lax.axis_index("c")
    it16 = _iota16()
    rows_lo = [it16 + g * 16 for g in range(4)]

    def fire_load(i, s_v, sem):
        rt = wid + NW * i

        @pl.when(rt < RT_FULL)
        def _():
            off = pl.multiple_of(rt * 128, 128)
            pltpu.async_copy(tv_hbm.at[:, pl.ds(off, 128)], s_v, sem)

    def wait_load(i, s_v, sem):
        rt = wid + NW * i

        @pl.when(rt < RT_FULL)
        def _():
            pltpu.make_async_copy(tv_hbm.at[:, pl.ds(0, 128)], s_v, sem).wait()

    def transpose(s_v, o_v, nq):
        def body(q8, carry):
            for dq in range(8):
                q = q8 * 8 + dq
                ce = jnp.broadcast_to(2 * q, (16,)).astype(jnp.int32)
                co = ce + 1
                for g in range(4):
                    o_v[q, pl.ds(g * 16, 16)] = plsc.load_gather(
                        s_v, [rows_lo[g], ce])
                for g in range(4):
                    o_v[q, pl.ds(64 + g * 16, 16)] = plsc.load_gather(
                        s_v, [rows_lo[g], co])
            return carry

        lax.fori_loop(0, nq // 8, body, 0)

    def fire_store(i, o_v, sem):
        rt = wid + NW * i

        @pl.when(rt < RT_FULL)
        def _():
            off = pl.multiple_of(rt * 64, 64)
            pltpu.async_copy(o_v, out_hbm.at[pl.ds(off, 64)], sem)

    def wait_store(i, o_v, sem):
        rt = wid + NW * i

        @pl.when(rt < RT_FULL)
        def _():
            pltpu.make_async_copy(o_v, out_hbm.at[pl.ds(0, 64)], sem).wait()

    bufs = ((s0, o0, sem_i0, sem_o0), (s1, o1, sem_i1, sem_o1))
    fire_load(0, s0, sem_i0)
    fire_load(1, s1, sem_i1)

    def body(h, carry):
        for sub in range(2):
            i = 2 * h + sub
            s_v, o_v, sem_i, sem_o = bufs[sub]
            wait_load(i, s_v, sem_i)

            @pl.when(h >= 1)
            def _():
                wait_store(i - 2, o_v, sem_o)

            transpose(s_v, o_v, 64)
            fire_load(i + 2, s_v, sem_i)
            fire_store(i, o_v, sem_o)
        return carry

    lax.fori_loop(0, A_ITERS // 2, body, 0)
    wait_store(A_ITERS - 2, o0, sem_o0)
    wait_store(A_ITERS - 1, o1, sem_o1)

    @pl.when(wid == 0)
    def _():
        pltpu.sync_copy(tail_hbm, s0)
        transpose(s0, o0, 32)
        pltpu.sync_copy(o0.at[pl.ds(0, 32)],
                        out_hbm.at[pl.ds(PACKED_ROWS - 32, 32)])


@functools.partial(
    pl.kernel,
    mesh=_mesh,
    compiler_params=_params,
    out_type=jax.ShapeDtypeStruct((T, DIM, B), jnp.float32),
    scratch_types=[
        pltpu.VMEM((8, 128), jnp.int32),
        pltpu.VMEM((8, 128), jnp.int32),
        pltpu.VMEM((8, 128), jnp.int32),
        pltpu.VMEM((128, 128), jnp.float32),
        pltpu.VMEM((128, 128), jnp.float32),
        pltpu.VMEM((64, 128), jnp.float32),
        pltpu.VMEM((64, 128), jnp.float32),
        pltpu.SemaphoreType.DMA,
        pltpu.SemaphoreType.DMA,
        pltpu.SemaphoreType.DMA,
        pltpu.SemaphoreType.DMA,
    ],
)
def _gather_kernel(tab_hbm, idx_hbm, out_hbm, idx_v, p_v, h_v, r0, r1,
                   o0, o1, sem_g0, sem_g1, sem_w0, sem_w1):
    wid = lax.axis_index("s") * NC + lax.axis_index("c")
    boff = pl.multiple_of(wid * 128, 128)
    it16 = _iota16()
    rows_st = [it16 + g * 16 for g in range(8)]

    gbufs = ((r0, sem_g0), (r1, sem_g1))
    obufs = ((o0, sem_w0), (o1, sem_w1))

    def fire_gather(ts, r_v, sem):
        pltpu.async_copy(tab_hbm.at[p_v.at[ts]], r_v, sem)

    def wait_gather(r_v, sem):
        pltpu.make_async_copy(tab_hbm.at[pl.ds(0, 128)], r_v, sem).wait()

    def fire_write(t, o_v, sem):
        pltpu.async_copy(o_v, out_hbm.at[t, :, pl.ds(boff, 128)], sem)

    def wait_write(o_v, sem):
        pltpu.make_async_copy(o_v, out_hbm.at[0, :, pl.ds(0, 128)], sem).wait()

    def prep_tile(tt):
        toff = pl.multiple_of(tt * 8, 8)
        pltpu.sync_copy(idx_hbm.at[pl.ds(toff, 8), pl.ds(boff, 128)], idx_v)
        for ts in range(8):
            for g in range(8):
                v = idx_v[ts, pl.ds(g * 16, 16)]
                p_v[ts, pl.ds(g * 16, 16)] = lax.shift_right_logical(v, 1)
                h_v[ts, pl.ds(g * 16, 16)] = lax.shift_left(v & 1, 6)

    def transpose(ts, r_v, o_v):
        hb = [h_v[ts, pl.ds(g * 16, 16)] for g in range(8)]

        def body(c8, carry):
            for dc in range(8):
                c = c8 * 8 + dc
                cb = jnp.broadcast_to(c, (16,)).astype(jnp.int32)
                for g in range(8):
                    o_v[c, pl.ds(g * 16, 16)] = plsc.load_gather(
                        r_v, [rows_st[g], hb[g] + cb])
            return carry

        lax.fori_loop(0, DIM // 8, body, 0)

    def tile_body(tt, carry):
        prep_tile(tt)
        fire_gather(0, r0, sem_g0)

        def ts_body(ts, carry2):
            for sub in range(2):
                t = tt * 8 + 2 * ts + sub
                r_v, sem_g = gbufs[sub]
                o_v, sem_w = obufs[sub]
                wait_gather(r_v, sem_g)

                @pl.when(2 * ts + sub < 7)
                def _():
                    nr, nsem = gbufs[1 - sub]
                    fire_gather(2 * ts + sub + 1, nr, nsem)

                @pl.when(tt * 8 + 2 * ts + sub >= 2)
                def _():
                    wait_write(o_v, sem_w)

                transpose(2 * ts + sub, r_v, o_v)
                fire_write(t, o_v, sem_w)
            return carry2

        lax.fori_loop(0, 4, ts_body, 0)
        return carry

    lax.fori_loop(0, T // 8, tile_body, 0)
    wait_write(o0, sem_w0)
    wait_write(o1, sem_w1)


def kernel(inputs, table):
    tv = table.T
    tail = jnp.pad(table[TAIL0:].T, ((0, 0), (0, 128 - (NUM_ROWS - TAIL0))))
    packed = _pack_kernel(tv, tail)
    out_t = _gather_kernel(packed, inputs.T.astype(jnp.int32))
    return out_t.transpose(2, 0, 1)


# speedup vs baseline: 1.8613x; 1.8613x over previous
import functools

import jax
import jax.numpy as jnp
from jax import lax
from jax.experimental import pallas as pl
from jax.experimental.pallas import tpu as pltpu
from jax.experimental.pallas import tpu_sc as plsc

NUM_ROWS = 1_000_000
DIM = 64
B, T = 4096, 200

NC, NS = 2, 16
NW = NC * NS
RT_FULL = NUM_ROWS // 128
A_ITERS = 246
PACKED_ROWS = NUM_ROWS // 2
TAIL0 = RT_FULL * 128

_mesh = plsc.VectorSubcoreMesh(core_axis_name="c", subcore_axis_name="s")
_params = pltpu.CompilerParams(use_tc_tiling_on_sc=True,
                               needs_layout_passes=False)


def _iota16():
    return jax.lax.iota(jnp.int32, 16)


@functools.partial(
    pl.kernel,
    mesh=_mesh,
    compiler_params=_params,
    out_type=jax.ShapeDtypeStruct((PACKED_ROWS, 128), jnp.float32),
    scratch_types=[
        pltpu.VMEM((64, 128), jnp.float32),
        pltpu.VMEM((64, 128), jnp.float32),
        pltpu.VMEM((64, 128), jnp.float32),
        pltpu.VMEM((64, 128), jnp.float32),
        pltpu.SemaphoreType.DMA,
        pltpu.SemaphoreType.DMA,
        pltpu.SemaphoreType.DMA,
        pltpu.SemaphoreType.DMA,
    ],
)
def _pack_kernel(tv_hbm, tail_hbm, out_hbm, s0, s1, o0, o1,
                 sem_i0, sem_i1, sem_o0, sem_o1):
    wid = lax.axis_index("s") * NC + lax.axis_index("c")
    it16 = _iota16()
    rows_lo = [it16 + g * 16 for g in range(4)]

    def fire_load(i, s_v, sem):
        rt = wid + NW * i

        @pl.when(rt < RT_FULL)
        def _():
            off = pl.multiple_of(rt * 128, 128)
            pltpu.async_copy(tv_hbm.at[:, pl.ds(off, 128)], s_v, sem)

    def wait_load(i, s_v, sem):
        rt = wid + NW * i

        @pl.when(rt < RT_FULL)
        def _():
            pltpu.make_async_copy(tv_hbm.at[:, pl.ds(0, 128)], s_v, sem).wait()

    def transpose(s_v, o_v, nq):
        @plsc.parallel_loop(0, nq, step=1, unroll=8)
        def _(q):
            ce = jnp.broadcast_to(2 * q, (16,)).astype(jnp.int32)
            co = ce + 1
            for g in range(4):
                o_v[q, pl.ds(g * 16, 16)] = plsc.load_gather(
                    s_v, [rows_lo[g], ce])
            for g in range(4):
                o_v[q, pl.ds(64 + g * 16, 16)] = plsc.load_gather(
                    s_v, [rows_lo[g], co])

    def fire_store(i, o_v, sem):
        rt = wid + NW * i

        @pl.when(rt < RT_FULL)
        def _():
            off = pl.multiple_of(rt * 64, 64)
            pltpu.async_copy(o_v, out_hbm.at[pl.ds(off, 64)], sem)

    def wait_store(i, o_v, sem):
        rt = wid + NW * i

        @pl.when(rt < RT_FULL)
        def _():
            pltpu.make_async_copy(o_v, out_hbm.at[pl.ds(0, 64)], sem).wait()

    bufs = ((s0, o0, sem_i0, sem_o0), (s1, o1, sem_i1, sem_o1))
    fire_load(0, s0, sem_i0)
    fire_load(1, s1, sem_i1)

    def body(h, carry):
        for sub in range(2):
            i = 2 * h + sub
            s_v, o_v, sem_i, sem_o = bufs[sub]
            wait_load(i, s_v, sem_i)

            @pl.when(h >= 1)
            def _():
                wait_store(i - 2, o_v, sem_o)

            transpose(s_v, o_v, 64)
            fire_load(i + 2, s_v, sem_i)
            fire_store(i, o_v, sem_o)
        return carry

    lax.fori_loop(0, A_ITERS // 2, body, 0)
    wait_store(A_ITERS - 2, o0, sem_o0)
    wait_store(A_ITERS - 1, o1, sem_o1)

    @pl.when(wid == 0)
    def _():
        pltpu.sync_copy(tail_hbm, s0)
        transpose(s0, o0, 32)
        pltpu.sync_copy(o0.at[pl.ds(0, 32)],
                        out_hbm.at[pl.ds(PACKED_ROWS - 32, 32)])


@functools.partial(
    pl.kernel,
    mesh=_mesh,
    compiler_params=_params,
    out_type=jax.ShapeDtypeStruct((T, DIM, B), jnp.float32),
    scratch_types=[
        pltpu.VMEM((8, 128), jnp.int32),
        pltpu.VMEM((8, 128), jnp.int32),
        pltpu.VMEM((8, 128), jnp.int32),
        pltpu.VMEM((128, 128), jnp.float32),
        pltpu.VMEM((128, 128), jnp.float32),
        pltpu.VMEM((64, 128), jnp.float32),
        pltpu.VMEM((64, 128), jnp.float32),
        pltpu.SemaphoreType.DMA,
        pltpu.SemaphoreType.DMA,
        pltpu.SemaphoreType.DMA,
        pltpu.SemaphoreType.DMA,
    ],
)
def _gather_kernel(tab_hbm, idx_hbm, out_hbm, idx_v, p_v, h_v, r0, r1,
                   o0, o1, sem_g0, sem_g1, sem_w0, sem_w1):
    wid = lax.axis_index("s") * NC + lax.axis_index("c")
    boff = pl.multiple_of(wid * 128, 128)
    it16 = _iota16()
    rows_st = [it16 + g * 16 for g in range(8)]

    gbufs = ((r0, sem_g0), (r1, sem_g1))
    obufs = ((o0, sem_w0), (o1, sem_w1))

    def fire_gather(ts, r_v, sem):
        pltpu.async_copy(tab_hbm.at[p_v.at[ts]], r_v, sem)

    def wait_gather(r_v, sem):
        pltpu.make_async_copy(tab_hbm.at[pl.ds(0, 128)], r_v, sem).wait()

    def fire_write(t, o_v, sem):
        pltpu.async_copy(o_v, out_hbm.at[t, :, pl.ds(boff, 128)], sem)

    def wait_write(o_v, sem):
        pltpu.make_async_copy(o_v, out_hbm.at[0, :, pl.ds(0, 128)], sem).wait()

    def prep_tile(tt):
        toff = pl.multiple_of(tt * 8, 8)
        pltpu.sync_copy(idx_hbm.at[pl.ds(toff, 8), pl.ds(boff, 128)], idx_v)
        for ts in range(8):
            for g in range(8):
                v = idx_v[ts, pl.ds(g * 16, 16)]
                p_v[ts, pl.ds(g * 16, 16)] = lax.shift_right_logical(v, 1)
                h_v[ts, pl.ds(g * 16, 16)] = lax.shift_left(v & 1, 6)

    def transpose(ts, r_v, o_v):
        hb = [h_v[ts, pl.ds(g * 16, 16)] for g in range(8)]

        @plsc.parallel_loop(0, DIM, step=1, unroll=8)
        def _(c):
            cb = jnp.broadcast_to(c, (16,)).astype(jnp.int32)
            for g in range(8):
                o_v[c, pl.ds(g * 16, 16)] = plsc.load_gather(
                    r_v, [rows_st[g], hb[g] + cb])

    def tile_body(tt, carry):
        prep_tile(tt)
        fire_gather(0, r0, sem_g0)

        def ts_body(ts, carry2):
            for sub in range(2):
                t = tt * 8 + 2 * ts + sub
                r_v, sem_g = gbufs[sub]
                o_v, sem_w = obufs[sub]
                wait_gather(r_v, sem_g)

                @pl.when(2 * ts + sub < 7)
                def _():
                    nr, nsem = gbufs[1 - sub]
                    fire_gather(2 * ts + sub + 1, nr, nsem)

                @pl.when(tt * 8 + 2 * ts + sub >= 2)
                def _():
                    wait_write(o_v, sem_w)

                transpose(2 * ts + sub, r_v, o_v)
                fire_write(t, o_v, sem_w)
            return carry2

        lax.fori_loop(0, 4, ts_body, 0)
        return carry

    lax.fori_loop(0, T // 8, tile_body, 0)
    wait_write(o0, sem_w0)
    wait_write(o1, sem_w1)


def kernel(inputs, table):
    tv = table.T
    tail = jnp.pad(table[TAIL0:].T, ((0, 0), (0, 128 - (NUM_ROWS - TAIL0))))
    packed = _pack_kernel(tv, tail)
    out_t = _gather_kernel(packed, inputs.T.astype(jnp.int32))
    return out_t.transpose(2, 0, 1)


# speedup vs baseline: 1.9022x; 1.0220x over previous
import functools

import jax
import jax.numpy as jnp
from jax import lax
from jax.experimental import pallas as pl
from jax.experimental.pallas import tpu as pltpu
from jax.experimental.pallas import tpu_sc as plsc

NUM_ROWS = 1_000_000
DIM = 64
B, T = 4096, 200

NC, NS = 2, 16
NW = NC * NS
RT_FULL = NUM_ROWS // 128
A_ITERS = 248
PACKED_ROWS = NUM_ROWS // 2
TAIL0 = RT_FULL * 128

_mesh = plsc.VectorSubcoreMesh(core_axis_name="c", subcore_axis_name="s")
_params = pltpu.CompilerParams(use_tc_tiling_on_sc=True,
                               needs_layout_passes=False)


def _iota16():
    return jax.lax.iota(jnp.int32, 16)


@functools.partial(
    pl.kernel,
    mesh=_mesh,
    compiler_params=_params,
    out_type=jax.ShapeDtypeStruct((PACKED_ROWS, 128), jnp.float32),
    scratch_types=[
        pltpu.VMEM((64, 128), jnp.float32),
        pltpu.VMEM((64, 128), jnp.float32),
        pltpu.VMEM((64, 128), jnp.float32),
        pltpu.VMEM((64, 128), jnp.float32),
        pltpu.VMEM((64, 128), jnp.float32),
        pltpu.VMEM((64, 128), jnp.float32),
        pltpu.SemaphoreType.DMA,
        pltpu.SemaphoreType.DMA,
        pltpu.SemaphoreType.DMA,
        pltpu.SemaphoreType.DMA,
        pltpu.SemaphoreType.DMA,
        pltpu.SemaphoreType.DMA,
    ],
)
def _pack_kernel(tv_hbm, tail_hbm, out_hbm, s0, s1, s2, s3, o0, o1,
                 si0, si1, si2, si3, so0, so1):
    wid = lax.axis_index("s") * NC + lax.axis_index("c")
    it16 = _iota16()
    rows_lo = [it16 + g * 16 for g in range(4)]
    sbufs = ((s0, si0), (s1, si1), (s2, si2), (s3, si3))
    obufs = ((o0, so0), (o1, so1))

    def fire_load(i, s_v, sem):
        rt = wid + NW * i

        @pl.when(rt < RT_FULL)
        def _():
            off = pl.multiple_of(rt * 128, 128)
            pltpu.async_copy(tv_hbm.at[:, pl.ds(off, 128)], s_v, sem)

    def wait_load(i, s_v, sem):
        rt = wid + NW * i

        @pl.when(rt < RT_FULL)
        def _():
            pltpu.make_async_copy(tv_hbm.at[:, pl.ds(0, 128)], s_v, sem).wait()

    def transpose(s_v, o_v, nq):
        @plsc.parallel_loop(0, nq, step=1, unroll=8)
        def _(q):
            ce = jnp.broadcast_to(2 * q, (16,)).astype(jnp.int32)
            co = ce + 1
            for g in range(4):
                o_v[q, pl.ds(g * 16, 16)] = plsc.load_gather(
                    s_v, [rows_lo[g], ce])
            for g in range(4):
                o_v[q, pl.ds(64 + g * 16, 16)] = plsc.load_gather(
                    s_v, [rows_lo[g], co])

    def fire_store(i, o_v, sem):
        rt = wid + NW * i

        @pl.when(rt < RT_FULL)
        def _():
            off = pl.multiple_of(rt * 64, 64)
            pltpu.async_copy(o_v, out_hbm.at[pl.ds(off, 64)], sem)

    def wait_store(i, o_v, sem):
        rt = wid + NW * i

        @pl.when(rt < RT_FULL)
        def _():
            pltpu.make_async_copy(o_v, out_hbm.at[pl.ds(0, 64)], sem).wait()

    for k in range(3):
        fire_load(k, sbufs[k][0], sbufs[k][1])

    def body(h, carry):
        for sub in range(4):
            i = 4 * h + sub
            s_v, sem_i = sbufs[sub]
            o_v, sem_o = obufs[sub % 2]
            wait_load(i, s_v, sem_i)

            @pl.when(4 * h + sub >= 2)
            def _():
                wait_store(i - 2, o_v, sem_o)

            transpose(s_v, o_v, 64)
            fire_load(i + 3, sbufs[(sub + 3) % 4][0], sbufs[(sub + 3) % 4][1])
            fire_store(i, o_v, sem_o)
        return carry

    lax.fori_loop(0, A_ITERS // 4, body, 0)
    wait_store(A_ITERS - 2, o0, so0)
    wait_store(A_ITERS - 1, o1, so1)

    @pl.when(wid == 0)
    def _():
        pltpu.sync_copy(tail_hbm, s0)
        transpose(s0, o0, 32)
        pltpu.sync_copy(o0.at[pl.ds(0, 32)],
                        out_hbm.at[pl.ds(PACKED_ROWS - 32, 32)])


@functools.partial(
    pl.kernel,
    mesh=_mesh,
    compiler_params=_params,
    out_type=jax.ShapeDtypeStruct((T, DIM, B), jnp.float32),
    scratch_types=[
        pltpu.VMEM((2, 8, 128), jnp.int32),
        pltpu.VMEM((2, 8, 128), jnp.int32),
        pltpu.VMEM((2, 8, 128), jnp.int32),
        pltpu.VMEM((128, 128), jnp.float32),
        pltpu.VMEM((128, 128), jnp.float32),
        pltpu.VMEM((128, 128), jnp.float32),
        pltpu.VMEM((128, 128), jnp.float32),
        pltpu.VMEM((64, 128), jnp.float32),
        pltpu.VMEM((64, 128), jnp.float32),
        pltpu.SemaphoreType.DMA,
        pltpu.SemaphoreType.DMA,
        pltpu.SemaphoreType.DMA,
        pltpu.SemaphoreType.DMA,
        pltpu.SemaphoreType.DMA,
        pltpu.SemaphoreType.DMA,
        pltpu.SemaphoreType.DMA,
    ],
)
def _gather_kernel(tab_hbm, idx_hbm, out_hbm, idx_v, p_v, h_v,
                   r0, r1, r2, r3, o0, o1,
                   sem_idx, sg0, sg1, sg2, sg3, sw0, sw1):
    wid = lax.axis_index("s") * NC + lax.axis_index("c")
    boff = pl.multiple_of(wid * 128, 128)
    it16 = _iota16()
    rows_st = [it16 + g * 16 for g in range(8)]
    gbufs = ((r0, sg0), (r1, sg1), (r2, sg2), (r3, sg3))
    obufs = ((o0, sw0), (o1, sw1))
    NT = T // 8

    def fire_idx(tt1):
        toff = pl.multiple_of(tt1 * 8, 8)
        pltpu.async_copy(idx_hbm.at[pl.ds(toff, 8), pl.ds(boff, 128)],
                         idx_v.at[tt1 & 1], sem_idx)

    def wait_idx():
        pltpu.make_async_copy(idx_hbm.at[pl.ds(0, 8), pl.ds(0, 128)],
                              idx_v.at[0], sem_idx).wait()

    def prep(slot):
        for ts in range(8):
            for g in range(8):
                v = idx_v[slot, ts, pl.ds(g * 16, 16)]
                p_v[slot, ts, pl.ds(g * 16, 16)] = lax.shift_right_logical(v, 1)
                h_v[slot, ts, pl.ds(g * 16, 16)] = lax.shift_left(v & 1, 6)

    def fire_gather(slot, ts, r_v, sem):
        pltpu.async_copy(tab_hbm.at[p_v.at[slot, ts]], r_v, sem)

    def wait_gather(r_v, sem):
        pltpu.make_async_copy(tab_hbm.at[pl.ds(0, 128)], r_v, sem).wait()

    def fire_write(t, o_v, sem):
        pltpu.async_copy(o_v, out_hbm.at[t, :, pl.ds(boff, 128)], sem)

    def wait_write(o_v, sem):
        pltpu.make_async_copy(o_v, out_hbm.at[0, :, pl.ds(0, 128)], sem).wait()

    def transpose(slot, ts, r_v, o_v):
        hb = [h_v[slot, ts, pl.ds(g * 16, 16)] for g in range(8)]

        @plsc.parallel_loop(0, DIM, step=1, unroll=8)
        def _(c):
            cb = jnp.broadcast_to(c, (16,)).astype(jnp.int32)
            for g in range(8):
                o_v[c, pl.ds(g * 16, 16)] = plsc.load_gather(
                    r_v, [rows_st[g], hb[g] + cb])

    pltpu.sync_copy(idx_hbm.at[pl.ds(0, 8), pl.ds(boff, 128)], idx_v.at[0])
    prep(0)
    for k in range(3):
        fire_gather(0, k, gbufs[k][0], gbufs[k][1])

    def tile_body(tt, carry):
        slot = tt & 1
        nslot = 1 - slot
        for ts in range(8):
            if ts == 0:
                @pl.when(tt < NT - 1)
                def _():
                    fire_idx(tt + 1)
            r_v, sem_g = gbufs[ts % 4]
            o_v, sem_w = obufs[ts % 2]
            wait_gather(r_v, sem_g)

            @pl.when(tt * 8 + ts >= 2)
            def _():
                wait_write(o_v, sem_w)

            transpose(slot, ts, r_v, o_v)
            fire_write(tt * 8 + ts, o_v, sem_w)
            if ts == 4:
                @pl.when(tt < NT - 1)
                def _():
                    wait_idx()
                    prep(nslot)
            if ts < 5:
                fire_gather(slot, ts + 3, gbufs[(ts + 3) % 4][0],
                            gbufs[(ts + 3) % 4][1])
            else:
                @pl.when(tt < NT - 1)
                def _():
                    fire_gather(nslot, ts - 5, gbufs[(ts + 3) % 4][0],
                                gbufs[(ts + 3) % 4][1])
        return carry

    lax.fori_loop(0, NT, tile_body, 0)
    wait_write(o0, sw0)
    wait_write(o1, sw1)


def kernel(inputs, table):
    tv = table.T
    tail = jnp.pad(table[TAIL0:].T, ((0, 0), (0, 128 - (NUM_ROWS - TAIL0))))
    packed = _pack_kernel(tv, tail)
    out_t = _gather_kernel(packed, inputs.T.astype(jnp.int32))
    return out_t.transpose(2, 0, 1)


# speedup vs baseline: 6.4810x; 3.4071x over previous
import functools

import jax
import jax.numpy as jnp
from jax import lax
from jax.experimental import pallas as pl
from jax.experimental.pallas import tpu as pltpu
from jax.experimental.pallas import tpu_sc as plsc

NUM_ROWS = 1_000_000
DIM = 64
B, T = 4096, 200

NC, NS = 2, 16
NW = NC * NS
RT_FULL = NUM_ROWS // 128
A_ITERS = 248
PACKED_ROWS = NUM_ROWS // 2
TAIL0 = RT_FULL * 128

_mesh = plsc.VectorSubcoreMesh(core_axis_name="c", subcore_axis_name="s")
_params = pltpu.CompilerParams(use_tc_tiling_on_sc=True,
                               needs_layout_passes=False)


def _iota16():
    return jax.lax.iota(jnp.int32, 16)


@functools.partial(
    pl.kernel,
    mesh=_mesh,
    compiler_params=_params,
    out_type=jax.ShapeDtypeStruct((PACKED_ROWS, 128), jnp.float32),
    scratch_types=[
        pltpu.VMEM((64, 128), jnp.float32),
        pltpu.VMEM((64, 128), jnp.float32),
        pltpu.VMEM((64, 128), jnp.float32),
        pltpu.VMEM((64, 128), jnp.float32),
        pltpu.VMEM((64, 128), jnp.float32),
        pltpu.VMEM((64, 128), jnp.float32),
        pltpu.SemaphoreType.DMA,
        pltpu.SemaphoreType.DMA,
        pltpu.SemaphoreType.DMA,
        pltpu.SemaphoreType.DMA,
        pltpu.SemaphoreType.DMA,
        pltpu.SemaphoreType.DMA,
    ],
)
def _pack_kernel(tv_hbm, tail_hbm, out_hbm, s0, s1, s2, s3, o0, o1,
                 si0, si1, si2, si3, so0, so1):
    wid = lax.axis_index("s") * NC + lax.axis_index("c")
    it16 = _iota16()
    rows8 = [it16 + g * 16 for g in range(8)]
    qs = [lax.shift_right_logical(r, 1) for r in rows8]
    j64 = [lax.shift_left(r & 1, 6) for r in rows8]
    sbufs = ((s0, si0), (s1, si1), (s2, si2), (s3, si3))
    obufs = ((o0, so0), (o1, so1))

    def fire_load(i, s_v, sem):
        rt = wid + NW * i

        @pl.when(rt < RT_FULL)
        def _():
            off = pl.multiple_of(rt * 128, 128)
            pltpu.async_copy(tv_hbm.at[:, pl.ds(off, 128)], s_v, sem)

    def wait_load(i, s_v, sem):
        rt = wid + NW * i

        @pl.when(rt < RT_FULL)
        def _():
            pltpu.make_async_copy(tv_hbm.at[:, pl.ds(0, 128)], s_v, sem).wait()

    def transpose(s_v, o_v):
        @plsc.parallel_loop(0, 64, step=1, unroll=8)
        def _(c0):
            c0b = jnp.broadcast_to(c0, (16,)).astype(jnp.int32)
            for g in range(8):
                c = (c0b + rows8[g]) & 63
                x = plsc.load_gather(s_v, [c, rows8[g]])
                plsc.store_scatter(o_v, [qs[g], c + j64[g]], x)

    def fire_store(i, o_v, sem):
        rt = wid + NW * i

        @pl.when(rt < RT_FULL)
        def _():
            off = pl.multiple_of(rt * 64, 64)
            pltpu.async_copy(o_v, out_hbm.at[pl.ds(off, 64)], sem)

    def wait_store(i, o_v, sem):
        rt = wid + NW * i

        @pl.when(rt < RT_FULL)
        def _():
            pltpu.make_async_copy(o_v, out_hbm.at[pl.ds(0, 64)], sem).wait()

    for k in range(3):
        fire_load(k, sbufs[k][0], sbufs[k][1])

    def body(h, carry):
        for sub in range(4):
            i = 4 * h + sub
            s_v, sem_i = sbufs[sub]
            o_v, sem_o = obufs[sub % 2]
            wait_load(i, s_v, sem_i)

            @pl.when(4 * h + sub >= 2)
            def _():
                wait_store(i - 2, o_v, sem_o)

            transpose(s_v, o_v)
            fire_load(i + 3, sbufs[(sub + 3) % 4][0], sbufs[(sub + 3) % 4][1])
            fire_store(i, o_v, sem_o)
        return carry

    lax.fori_loop(0, A_ITERS // 4, body, 0)
    wait_store(A_ITERS - 2, o0, so0)
    wait_store(A_ITERS - 1, o1, so1)

    @pl.when(wid == 0)
    def _():
        pltpu.sync_copy(tail_hbm, s0)
        transpose(s0, o0)
        pltpu.sync_copy(o0.at[pl.ds(0, 32)],
                        out_hbm.at[pl.ds(PACKED_ROWS - 32, 32)])


@functools.partial(
    pl.kernel,
    mesh=_mesh,
    compiler_params=_params,
    out_type=jax.ShapeDtypeStruct((T, DIM, B), jnp.float32),
    scratch_types=[
        pltpu.VMEM((2, 8, 128), jnp.int32),
        pltpu.VMEM((2, 8, 128), jnp.int32),
        pltpu.VMEM((2, 8, 128), jnp.int32),
        pltpu.VMEM((128, 128), jnp.float32),
        pltpu.VMEM((128, 128), jnp.float32),
        pltpu.VMEM((128, 128), jnp.float32),
        pltpu.VMEM((128, 128), jnp.float32),
        pltpu.VMEM((64, 128), jnp.float32),
        pltpu.VMEM((64, 128), jnp.float32),
        pltpu.SemaphoreType.DMA,
        pltpu.SemaphoreType.DMA,
        pltpu.SemaphoreType.DMA,
        pltpu.SemaphoreType.DMA,
        pltpu.SemaphoreType.DMA,
        pltpu.SemaphoreType.DMA,
        pltpu.SemaphoreType.DMA,
    ],
)
def _gather_kernel(tab_hbm, idx_hbm, out_hbm, idx_v, p_v, h_v,
                   r0, r1, r2, r3, o0, o1,
                   sem_idx, sg0, sg1, sg2, sg3, sw0, sw1):
    wid = lax.axis_index("s") * NC + lax.axis_index("c")
    boff = pl.multiple_of(wid * 128, 128)
    it16 = _iota16()
    rows_st = [it16 + g * 16 for g in range(8)]
    gbufs = ((r0, sg0), (r1, sg1), (r2, sg2), (r3, sg3))
    obufs = ((o0, sw0), (o1, sw1))
    NT = T // 8

    def fire_idx(tt1):
        toff = pl.multiple_of(tt1 * 8, 8)
        pltpu.async_copy(idx_hbm.at[pl.ds(toff, 8), pl.ds(boff, 128)],
                         idx_v.at[tt1 & 1], sem_idx)

    def wait_idx():
        pltpu.make_async_copy(idx_hbm.at[pl.ds(0, 8), pl.ds(0, 128)],
                              idx_v.at[0], sem_idx).wait()

    def prep(slot):
        for ts in range(8):
            for g in range(8):
                v = idx_v[slot, ts, pl.ds(g * 16, 16)]
                p_v[slot, ts, pl.ds(g * 16, 16)] = lax.shift_right_logical(v, 1)
                h_v[slot, ts, pl.ds(g * 16, 16)] = lax.shift_left(v & 1, 6)

    def fire_gather(slot, ts, r_v, sem):
        pltpu.async_copy(tab_hbm.at[p_v.at[slot, ts]], r_v, sem)

    def wait_gather(r_v, sem):
        pltpu.make_async_copy(tab_hbm.at[pl.ds(0, 128)], r_v, sem).wait()

    def fire_write(t, o_v, sem):
        pltpu.async_copy(o_v, out_hbm.at[t, :, pl.ds(boff, 128)], sem)

    def wait_write(o_v, sem):
        pltpu.make_async_copy(o_v, out_hbm.at[0, :, pl.ds(0, 128)], sem).wait()

    def transpose(slot, ts, r_v, o_v):
        hb = [h_v[slot, ts, pl.ds(g * 16, 16)] for g in range(8)]

        @plsc.parallel_loop(0, DIM, step=1, unroll=8)
        def _(k):
            kb = jnp.broadcast_to(k, (16,)).astype(jnp.int32)
            for g in range(8):
                c = (rows_st[g] + kb) & 63
                x = plsc.load_gather(r_v, [rows_st[g], hb[g] + c])
                plsc.store_scatter(o_v, [c, rows_st[g]], x)

    pltpu.sync_copy(idx_hbm.at[pl.ds(0, 8), pl.ds(boff, 128)], idx_v.at[0])
    prep(0)
    for k in range(3):
        fire_gather(0, k, gbufs[k][0], gbufs[k][1])

    def tile_body(tt, carry):
        slot = tt & 1
        nslot = 1 - slot
        for ts in range(8):
            if ts == 0:
                @pl.when(tt < NT - 1)
                def _():
                    fire_idx(tt + 1)
            r_v, sem_g = gbufs[ts % 4]
            o_v, sem_w = obufs[ts % 2]
            wait_gather(r_v, sem_g)

            @pl.when(tt * 8 + ts >= 2)
            def _():
                wait_write(o_v, sem_w)

            transpose(slot, ts, r_v, o_v)
            fire_write(tt * 8 + ts, o_v, sem_w)
            if ts == 4:
                @pl.when(tt < NT - 1)
                def _():
                    wait_idx()
                    prep(nslot)
            if ts < 5:
                fire_gather(slot, ts + 3, gbufs[(ts + 3) % 4][0],
                            gbufs[(ts + 3) % 4][1])
            else:
                @pl.when(tt < NT - 1)
                def _():
                    fire_gather(nslot, ts - 5, gbufs[(ts + 3) % 4][0],
                                gbufs[(ts + 3) % 4][1])
        return carry

    lax.fori_loop(0, NT, tile_body, 0)
    wait_write(o0, sw0)
    wait_write(o1, sw1)


def kernel(inputs, table):
    tv = table.T
    tail = jnp.pad(table[TAIL0:].T, ((0, 0), (0, 128 - (NUM_ROWS - TAIL0))))
    packed = _pack_kernel(tv, tail)
    out_t = _gather_kernel(packed, inputs.T.astype(jnp.int32))
    return out_t.transpose(2, 0, 1)
